# Initial kernel scaffold; baseline (speedup 1.0000x reference)
import jax
import jax.numpy as jnp
from jax.experimental import pallas as pl


def kernel(latents, embedding):
    raise NotImplementedError("write your pallas kernel here")


# speedup vs baseline: 7.1985x; 7.1985x over previous
import functools

import jax
import jax.numpy as jnp
from jax import lax
from jax.experimental import pallas as pl
from jax.experimental.pallas import tpu as pltpu
from jax.experimental.pallas import tpu_sc as plsc

K = 8192
D = 256
BETA = 0.25

N_TILE = 512
K_CHUNK = 2048
WINDOWS = ((0, 2736), (2736, 5472), (5472, 8192))
BIG = float("inf")


def _bf(x):
    u = lax.bitcast_convert_type(x, jnp.uint32)
    r = (u + jnp.uint32(0x7FFF) + ((u >> 16) & jnp.uint32(1))) & jnp.uint32(0xFFFF0000)
    return lax.bitcast_convert_type(r, jnp.float32)


def _vq_kernel(flat_ref, emb_ref, f2_ref, e2_ref, idx_ref, loss_ref, dist_ref):
    i = pl.program_id(0)

    @pl.when(i == 0)
    def _():
        loss_ref[0, 0] = 0.0

    flat = flat_ref[...]
    f2 = f2_ref[...]
    for j in range(K // K_CHUNK):
        emb_c = emb_ref[pl.ds(j * K_CHUNK, K_CHUNK), :]
        e2_c = e2_ref[:, pl.ds(j * K_CHUNK, K_CHUNK)]
        mm = lax.dot_general(flat, emb_c, (((1,), (1,)), ((), ())),
                             preferred_element_type=jnp.float32)
        dist_ref[:, pl.ds(j * K_CHUNK, K_CHUNK)] = (f2 + e2_c) - 2.0 * mm

    dist = dist_ref[...]
    iota = lax.broadcasted_iota(jnp.int32, (N_TILE, K), 1)
    wm, wa = [], []
    for (lo, hi) in WINDOWS:
        inwin = (iota >= lo) & (iota < hi)
        dw = jnp.where(inwin, dist, BIG)
        m = jnp.min(dw, axis=1, keepdims=True)
        a = jnp.min(jnp.where(dw == m, iota, jnp.int32(2 ** 30)),
                    axis=1, keepdims=True)
        wm.append(m)
        wa.append(a)
    m1, m2, m3 = wm
    a1, a2, a3 = wa

    b1 = _bf(m1)
    t2 = m2 < b1
    b2 = _bf(jnp.where(t2, m2, b1))
    t3 = m3 < b2

    use3 = t3
    use2 = t2 & ~t3
    idx = jnp.where(use3, a3, jnp.where(use2, a2, a1))
    lossv = jnp.where(use3, m3, jnp.where(use2, m2, m1))

    idx_ref[0, 0, :] = idx[:, 0]
    loss_ref[0, 0] += jnp.sum(lossv)


def _vq_call(flat, emb, f2, e2):
    n = flat.shape[0]
    return pl.pallas_call(
        _vq_kernel,
        grid=(n // N_TILE,),
        in_specs=[
            pl.BlockSpec((N_TILE, D), lambda i: (i, 0)),
            pl.BlockSpec((K, D), lambda i: (0, 0)),
            pl.BlockSpec((N_TILE, 1), lambda i: (i, 0)),
            pl.BlockSpec((1, K), lambda i: (0, 0)),
        ],
        out_specs=[
            pl.BlockSpec((1, 1, N_TILE), lambda i: (i, 0, 0)),
            pl.BlockSpec(memory_space=pltpu.SMEM),
        ],
        out_shape=[
            jax.ShapeDtypeStruct((n // N_TILE, 1, N_TILE), jnp.int32),
            jax.ShapeDtypeStruct((1, 1), jnp.float32),
        ],
        scratch_shapes=[pltpu.VMEM((N_TILE, K), jnp.float32)],
    )(flat, emb, f2, e2)


def _make_gather(n):
    info = plsc.get_sparse_core_info()
    nc, ns = info.num_cores, info.num_subcores
    nw = nc * ns
    b_per_w = n // nw
    chunk = 128
    nchunks = b_per_w // chunk
    mesh = plsc.VectorSubcoreMesh(core_axis_name="c", subcore_axis_name="s")

    @functools.partial(
        pl.kernel,
        mesh=mesh,
        out_type=jax.ShapeDtypeStruct((n, D), jnp.float32),
        scratch_types=[
            pltpu.VMEM((chunk,), jnp.int32),
            pltpu.VMEM((chunk, D), jnp.float32),
            pltpu.SemaphoreType.DMA,
        ],
    )
    def gather(table_hbm, idx_hbm, out_hbm, idx_v, rows_v, sem):
        wid = lax.axis_index("s") * nc + lax.axis_index("c")
        base = wid * b_per_w
        for c in range(nchunks):
            off = base + c * chunk
            pltpu.sync_copy(idx_hbm.at[pl.ds(off, chunk)], idx_v)
            pltpu.async_copy(table_hbm.at[idx_v], rows_v, sem).wait()
            pltpu.sync_copy(rows_v, out_hbm.at[pl.ds(off, chunk)])

    return gather


def kernel(latents, embedding):
    lat = jnp.transpose(latents, (0, 2, 3, 1))
    b, h, w, d = lat.shape
    flat = lat.reshape(-1, d)
    n = flat.shape[0]
    f2 = jnp.sum(flat ** 2, axis=1, keepdims=True)
    e2 = jnp.sum(embedding ** 2, axis=1)

    idx3, loss_sum = _vq_call(flat, embedding, f2, e2.reshape(1, K))
    idx = idx3.reshape(-1)

    quantized_flat = _make_gather(n)(embedding, idx)
    quantized_flat = quantized_flat.astype(jnp.bfloat16).astype(jnp.float32)
    quantized = quantized_flat.reshape(b, h, w, d)

    m = loss_sum[0, 0] / (n * d)
    vq_loss = m * BETA + m
    quantized_st = lat + (quantized - lat)
    return (jnp.transpose(quantized_st, (0, 3, 1, 2)), vq_loss)


# speedup vs baseline: 8.3256x; 1.1566x over previous
import functools

import jax
import jax.numpy as jnp
from jax import lax
from jax.experimental import pallas as pl
from jax.experimental.pallas import tpu as pltpu
from jax.experimental.pallas import tpu_sc as plsc

K = 8192
D = 256
BETA = 0.25

N_TILE = 512
K_CHUNK = 2048
WINDOWS = ((0, 2736), (2736, 5472), (5472, 8192))
BIG = float("inf")


def _bf(x):
    u = lax.bitcast_convert_type(x, jnp.uint32)
    r = (u + jnp.uint32(0x7FFF) + ((u >> 16) & jnp.uint32(1))) & jnp.uint32(0xFFFF0000)
    return lax.bitcast_convert_type(r, jnp.float32)


def _vq_kernel(flat_ref, emb_ref, f2_ref, e2_ref, idx_ref, loss_ref):
    i = pl.program_id(0)

    @pl.when(i == 0)
    def _():
        loss_ref[0, 0] = 0.0

    flat = flat_ref[...]
    f2 = f2_ref[...]
    wm = [jnp.full((N_TILE, 1), BIG, jnp.float32) for _ in WINDOWS]
    wa = [jnp.zeros((N_TILE, 1), jnp.int32) for _ in WINDOWS]
    for j in range(K // K_CHUNK):
        lo_c = j * K_CHUNK
        hi_c = lo_c + K_CHUNK
        emb_c = emb_ref[pl.ds(lo_c, K_CHUNK), :]
        e2_c = e2_ref[:, pl.ds(lo_c, K_CHUNK)]
        mm = lax.dot_general(flat, emb_c, (((1,), (1,)), ((), ())),
                             preferred_element_type=jnp.float32)
        dist = (f2 + e2_c) - 2.0 * mm
        iota = lax.broadcasted_iota(jnp.int32, dist.shape, 1) + lo_c
        for w, (lo, hi) in enumerate(WINDOWS):
            if hi <= lo_c or lo >= hi_c:
                continue
            if lo <= lo_c and hi >= hi_c:
                dw = dist
            else:
                inwin = (iota >= lo) & (iota < hi)
                dw = jnp.where(inwin, dist, BIG)
            m = jnp.min(dw, axis=1, keepdims=True)
            a = jnp.min(jnp.where(dw == m, iota, jnp.int32(2 ** 30)),
                        axis=1, keepdims=True)
            upd = m < wm[w]
            wa[w] = jnp.where(upd, a, wa[w])
            wm[w] = jnp.minimum(m, wm[w])
    m1, m2, m3 = wm
    a1, a2, a3 = wa

    b1 = _bf(m1)
    t2 = m2 < b1
    b2 = _bf(jnp.where(t2, m2, b1))
    t3 = m3 < b2

    use3 = t3
    use2 = t2 & ~t3
    idx = jnp.where(use3, a3, jnp.where(use2, a2, a1))
    lossv = jnp.where(use3, m3, jnp.where(use2, m2, m1))

    idx_ref[0, 0, :] = idx[:, 0]
    loss_ref[0, 0] += jnp.sum(lossv)


def _vq_call(flat, emb, f2, e2):
    n = flat.shape[0]
    return pl.pallas_call(
        _vq_kernel,
        grid=(n // N_TILE,),
        in_specs=[
            pl.BlockSpec((N_TILE, D), lambda i: (i, 0)),
            pl.BlockSpec((K, D), lambda i: (0, 0)),
            pl.BlockSpec((N_TILE, 1), lambda i: (i, 0)),
            pl.BlockSpec((1, K), lambda i: (0, 0)),
        ],
        out_specs=[
            pl.BlockSpec((1, 1, N_TILE), lambda i: (i, 0, 0)),
            pl.BlockSpec(memory_space=pltpu.SMEM),
        ],
        out_shape=[
            jax.ShapeDtypeStruct((n // N_TILE, 1, N_TILE), jnp.int32),
            jax.ShapeDtypeStruct((1, 1), jnp.float32),
        ],
    )(flat, emb, f2, e2)


def _make_gather(n):
    info = plsc.get_sparse_core_info()
    nc, ns = info.num_cores, info.num_subcores
    nw = nc * ns
    b_per_w = n // nw
    chunk = 128
    nchunks = b_per_w // chunk
    mesh = plsc.VectorSubcoreMesh(core_axis_name="c", subcore_axis_name="s")

    @functools.partial(
        pl.kernel,
        mesh=mesh,
        out_type=jax.ShapeDtypeStruct((n, D), jnp.float32),
        scratch_types=[
            pltpu.VMEM((chunk,), jnp.int32),
            pltpu.VMEM((chunk, D), jnp.float32),
            pltpu.SemaphoreType.DMA,
        ],
    )
    def gather(table_hbm, idx_hbm, out_hbm, idx_v, rows_v, sem):
        wid = lax.axis_index("s") * nc + lax.axis_index("c")
        base = wid * b_per_w
        for c in range(nchunks):
            off = base + c * chunk
            pltpu.sync_copy(idx_hbm.at[pl.ds(off, chunk)], idx_v)
            pltpu.async_copy(table_hbm.at[idx_v], rows_v, sem).wait()
            pltpu.sync_copy(rows_v, out_hbm.at[pl.ds(off, chunk)])

    return gather


def kernel(latents, embedding):
    lat = jnp.transpose(latents, (0, 2, 3, 1))
    b, h, w, d = lat.shape
    flat = lat.reshape(-1, d)
    n = flat.shape[0]
    f2 = jnp.sum(flat ** 2, axis=1, keepdims=True)
    e2 = jnp.sum(embedding ** 2, axis=1)

    idx3, loss_sum = _vq_call(flat, embedding, f2, e2.reshape(1, K))
    idx = idx3.reshape(-1)

    quantized_flat = _make_gather(n)(embedding, idx)
    quantized_flat = quantized_flat.astype(jnp.bfloat16).astype(jnp.float32)
    quantized = quantized_flat.reshape(b, h, w, d)

    m = loss_sum[0, 0] / (n * d)
    vq_loss = m * BETA + m
    quantized_st = lat + (quantized - lat)
    return (jnp.transpose(quantized_st, (0, 3, 1, 2)), vq_loss)


# speedup vs baseline: 8.8431x; 1.0622x over previous
import functools

import jax
import jax.numpy as jnp
from jax import lax
from jax.experimental import pallas as pl
from jax.experimental.pallas import tpu as pltpu
from jax.experimental.pallas import tpu_sc as plsc

K = 8192
D = 256
BETA = 0.25

N_TILE = 1024
K_CHUNK = 2048
WINDOWS = ((0, 2736), (2736, 5472), (5472, 8192))
BIG = float("inf")


def _bf(x):
    u = lax.bitcast_convert_type(x, jnp.uint32)
    r = (u + jnp.uint32(0x7FFF) + ((u >> 16) & jnp.uint32(1))) & jnp.uint32(0xFFFF0000)
    return lax.bitcast_convert_type(r, jnp.float32)


def _vq_kernel(flat_ref, emb_ref, f2_ref, e2_ref, idx_ref, loss_ref):
    i = pl.program_id(0)

    @pl.when(i == 0)
    def _():
        loss_ref[0, 0] = 0.0

    flat = flat_ref[...]
    f2 = f2_ref[...]
    wm = [jnp.full((N_TILE, 1), BIG, jnp.float32) for _ in WINDOWS]
    wa = [jnp.zeros((N_TILE, 1), jnp.int32) for _ in WINDOWS]
    for j in range(K // K_CHUNK):
        lo_c = j * K_CHUNK
        hi_c = lo_c + K_CHUNK
        emb_c = emb_ref[pl.ds(lo_c, K_CHUNK), :]
        e2_c = e2_ref[:, pl.ds(lo_c, K_CHUNK)]
        mm = lax.dot_general(flat, emb_c, (((1,), (1,)), ((), ())),
                             preferred_element_type=jnp.float32)
        dist = (f2 + e2_c) - 2.0 * mm
        iota = lax.broadcasted_iota(jnp.int32, dist.shape, 1) + lo_c
        for w, (lo, hi) in enumerate(WINDOWS):
            if hi <= lo_c or lo >= hi_c:
                continue
            if lo <= lo_c and hi >= hi_c:
                dw = dist
            else:
                inwin = (iota >= lo) & (iota < hi)
                dw = jnp.where(inwin, dist, BIG)
            m = jnp.min(dw, axis=1, keepdims=True)
            a = jnp.min(jnp.where(dw == m, iota, jnp.int32(2 ** 30)),
                        axis=1, keepdims=True)
            upd = m < wm[w]
            wa[w] = jnp.where(upd, a, wa[w])
            wm[w] = jnp.minimum(m, wm[w])
    m1, m2, m3 = wm
    a1, a2, a3 = wa

    b1 = _bf(m1)
    t2 = m2 < b1
    b2 = _bf(jnp.where(t2, m2, b1))
    t3 = m3 < b2

    use3 = t3
    use2 = t2 & ~t3
    idx = jnp.where(use3, a3, jnp.where(use2, a2, a1))
    lossv = jnp.where(use3, m3, jnp.where(use2, m2, m1))

    idx_ref[0, 0, :] = idx[:, 0]
    loss_ref[0, 0] += jnp.sum(lossv)


def _vq_call(flat, emb, f2, e2):
    n = flat.shape[0]
    return pl.pallas_call(
        _vq_kernel,
        grid=(n // N_TILE,),
        in_specs=[
            pl.BlockSpec((N_TILE, D), lambda i: (i, 0)),
            pl.BlockSpec((K, D), lambda i: (0, 0)),
            pl.BlockSpec((N_TILE, 1), lambda i: (i, 0)),
            pl.BlockSpec((1, K), lambda i: (0, 0)),
        ],
        out_specs=[
            pl.BlockSpec((1, 1, N_TILE), lambda i: (i, 0, 0)),
            pl.BlockSpec(memory_space=pltpu.SMEM),
        ],
        out_shape=[
            jax.ShapeDtypeStruct((n // N_TILE, 1, N_TILE), jnp.int32),
            jax.ShapeDtypeStruct((1, 1), jnp.float32),
        ],
    )(flat, emb, f2, e2)


def _make_gather(n):
    info = plsc.get_sparse_core_info()
    nc, ns = info.num_cores, info.num_subcores
    nw = nc * ns
    b_per_w = n // nw
    chunk = 128
    nchunks = b_per_w // chunk
    mesh = plsc.VectorSubcoreMesh(core_axis_name="c", subcore_axis_name="s")

    @functools.partial(
        pl.kernel,
        mesh=mesh,
        out_type=jax.ShapeDtypeStruct((n, D), jnp.float32),
        scratch_types=[
            pltpu.VMEM((chunk,), jnp.int32),
            pltpu.VMEM((chunk, D), jnp.float32),
            pltpu.SemaphoreType.DMA,
        ],
    )
    def gather(table_hbm, idx_hbm, out_hbm, idx_v, rows_v, sem):
        wid = lax.axis_index("s") * nc + lax.axis_index("c")
        base = wid * b_per_w
        for c in range(nchunks):
            off = base + c * chunk
            pltpu.sync_copy(idx_hbm.at[pl.ds(off, chunk)], idx_v)
            pltpu.async_copy(table_hbm.at[idx_v], rows_v, sem).wait()
            pltpu.sync_copy(rows_v, out_hbm.at[pl.ds(off, chunk)])

    return gather


def kernel(latents, embedding):
    lat = jnp.transpose(latents, (0, 2, 3, 1))
    b, h, w, d = lat.shape
    flat = lat.reshape(-1, d)
    n = flat.shape[0]
    f2 = jnp.sum(flat ** 2, axis=1, keepdims=True)
    e2 = jnp.sum(embedding ** 2, axis=1)

    idx3, loss_sum = _vq_call(flat, embedding, f2, e2.reshape(1, K))
    idx = idx3.reshape(-1)

    quantized_flat = _make_gather(n)(embedding, idx)
    quantized_flat = quantized_flat.astype(jnp.bfloat16).astype(jnp.float32)
    quantized = quantized_flat.reshape(b, h, w, d)

    m = loss_sum[0, 0] / (n * d)
    vq_loss = m * BETA + m
    quantized_st = lat + (quantized - lat)
    return (jnp.transpose(quantized_st, (0, 3, 1, 2)), vq_loss)


# speedup vs baseline: 9.3594x; 1.0584x over previous
import functools

import jax
import jax.numpy as jnp
from jax import lax
from jax.experimental import pallas as pl
from jax.experimental.pallas import tpu as pltpu
from jax.experimental.pallas import tpu_sc as plsc

K = 8192
D = 256
BETA = 0.25

N_TILE = 2048
K_CHUNK = 2048
WINDOWS = ((0, 2736), (2736, 5472), (5472, 8192))
BIG = float("inf")


def _bf(x):
    u = lax.bitcast_convert_type(x, jnp.uint32)
    r = (u + jnp.uint32(0x7FFF) + ((u >> 16) & jnp.uint32(1))) & jnp.uint32(0xFFFF0000)
    return lax.bitcast_convert_type(r, jnp.float32)


def _vq_kernel(flat_ref, emb_ref, f2_ref, e2_ref, idx_ref, loss_ref):
    i = pl.program_id(0)

    @pl.when(i == 0)
    def _():
        loss_ref[0, 0] = 0.0

    flat = flat_ref[...]
    f2 = f2_ref[...]
    wm = [jnp.full((N_TILE, 1), BIG, jnp.float32) for _ in WINDOWS]
    wa = [jnp.zeros((N_TILE, 1), jnp.int32) for _ in WINDOWS]
    for j in range(K // K_CHUNK):
        lo_c = j * K_CHUNK
        hi_c = lo_c + K_CHUNK
        emb_c = emb_ref[pl.ds(lo_c, K_CHUNK), :]
        e2_c = e2_ref[:, pl.ds(lo_c, K_CHUNK)]
        mm = lax.dot_general(flat, emb_c, (((1,), (1,)), ((), ())),
                             preferred_element_type=jnp.float32)
        dist = (f2 + e2_c) - 2.0 * mm
        iota = lax.broadcasted_iota(jnp.int32, dist.shape, 1) + lo_c
        for w, (lo, hi) in enumerate(WINDOWS):
            if hi <= lo_c or lo >= hi_c:
                continue
            if lo <= lo_c and hi >= hi_c:
                dw = dist
            else:
                inwin = (iota >= lo) & (iota < hi)
                dw = jnp.where(inwin, dist, BIG)
            m = jnp.min(dw, axis=1, keepdims=True)
            a = jnp.min(jnp.where(dw == m, iota, jnp.int32(2 ** 30)),
                        axis=1, keepdims=True)
            upd = m < wm[w]
            wa[w] = jnp.where(upd, a, wa[w])
            wm[w] = jnp.minimum(m, wm[w])
    m1, m2, m3 = wm
    a1, a2, a3 = wa

    b1 = _bf(m1)
    t2 = m2 < b1
    b2 = _bf(jnp.where(t2, m2, b1))
    t3 = m3 < b2

    use3 = t3
    use2 = t2 & ~t3
    idx = jnp.where(use3, a3, jnp.where(use2, a2, a1))
    lossv = jnp.where(use3, m3, jnp.where(use2, m2, m1))

    idx_ref[0, 0, :] = idx[:, 0]
    loss_ref[0, 0] += jnp.sum(lossv)


def _vq_call(flat, emb, f2, e2):
    n = flat.shape[0]
    return pl.pallas_call(
        _vq_kernel,
        grid=(n // N_TILE,),
        in_specs=[
            pl.BlockSpec((N_TILE, D), lambda i: (i, 0)),
            pl.BlockSpec((K, D), lambda i: (0, 0)),
            pl.BlockSpec((N_TILE, 1), lambda i: (i, 0)),
            pl.BlockSpec((1, K), lambda i: (0, 0)),
        ],
        out_specs=[
            pl.BlockSpec((1, 1, N_TILE), lambda i: (i, 0, 0)),
            pl.BlockSpec(memory_space=pltpu.SMEM),
        ],
        out_shape=[
            jax.ShapeDtypeStruct((n // N_TILE, 1, N_TILE), jnp.int32),
            jax.ShapeDtypeStruct((1, 1), jnp.float32),
        ],
    )(flat, emb, f2, e2)


def _make_gather(n):
    info = plsc.get_sparse_core_info()
    nc, ns = info.num_cores, info.num_subcores
    nw = nc * ns
    b_per_w = n // nw
    chunk = 128
    nchunks = b_per_w // chunk
    mesh = plsc.VectorSubcoreMesh(core_axis_name="c", subcore_axis_name="s")

    @functools.partial(
        pl.kernel,
        mesh=mesh,
        out_type=jax.ShapeDtypeStruct((n, D), jnp.float32),
        scratch_types=[
            pltpu.VMEM((chunk,), jnp.int32),
            pltpu.VMEM((chunk, D), jnp.float32),
            pltpu.SemaphoreType.DMA,
        ],
    )
    def gather(table_hbm, idx_hbm, out_hbm, idx_v, rows_v, sem):
        wid = lax.axis_index("s") * nc + lax.axis_index("c")
        base = wid * b_per_w
        for c in range(nchunks):
            off = base + c * chunk
            pltpu.sync_copy(idx_hbm.at[pl.ds(off, chunk)], idx_v)
            pltpu.async_copy(table_hbm.at[idx_v], rows_v, sem).wait()
            pltpu.sync_copy(rows_v, out_hbm.at[pl.ds(off, chunk)])

    return gather


def kernel(latents, embedding):
    lat = jnp.transpose(latents, (0, 2, 3, 1))
    b, h, w, d = lat.shape
    flat = lat.reshape(-1, d)
    n = flat.shape[0]
    f2 = jnp.sum(flat ** 2, axis=1, keepdims=True)
    e2 = jnp.sum(embedding ** 2, axis=1)

    idx3, loss_sum = _vq_call(flat, embedding, f2, e2.reshape(1, K))
    idx = idx3.reshape(-1)

    quantized_flat = _make_gather(n)(embedding, idx)
    quantized_flat = quantized_flat.astype(jnp.bfloat16).astype(jnp.float32)
    quantized = quantized_flat.reshape(b, h, w, d)

    m = loss_sum[0, 0] / (n * d)
    vq_loss = m * BETA + m
    quantized_st = lat + (quantized - lat)
    return (jnp.transpose(quantized_st, (0, 3, 1, 2)), vq_loss)
